# Initial kernel scaffold; baseline (speedup 1.0000x reference)
import jax
import jax.numpy as jnp
from jax.experimental import pallas as pl


def kernel(rec_z_buf, synapse_indices, weight_values, synaptic_basis_weights, syn_ids):
    raise NotImplementedError("write your pallas kernel here")


# speedup vs baseline: 11.8023x; 11.8023x over previous
import functools

import jax
import jax.numpy as jnp
from jax import lax
from jax.experimental import pallas as pl
from jax.experimental.pallas import tpu as pltpu
from jax.experimental.pallas import tpu_sc as plsc

N_PRE = 100000
N_POST = 100000
N_SYN = 6400000
N_BASIS = 5

NC = 2
NS = 16
NW = NC * NS

PER_W = N_SYN // NW
C = 2000
NCHUNK = PER_W // C
VPC = C // 16

NWORDS = 3200

STRIPE = 6256
LAST_START = N_POST - STRIPE


def _pack_body(z_ref, o_ref):
    shifts = lax.broadcasted_iota(jnp.int32, (NWORDS, 32), 1)
    bits = jnp.where(z_ref[:] > 0.0, lax.shift_left(jnp.int32(1), shifts), 0)
    o_ref[:] = jnp.sum(bits, axis=1)


def _sc_body(zbits_hbm, syn_hbm, w_hbm, basis_hbm, sid_hbm, zeros_hbm,
             out_hbm, acc, zbits_v, basis_v, stage_v, w_v, sid_v, post_v,
             vals_v):
    c = lax.axis_index("c")
    s = lax.axis_index("s")
    wid = s * NC + c
    r0 = pl.multiple_of(jnp.where(s == NS - 1, LAST_START, s * STRIPE), 8)

    pltpu.sync_copy(zeros_hbm.at[pl.ds(r0, STRIPE)],
                    acc.at[pl.ds(r0, STRIPE)])
    pltpu.sync_copy(zbits_hbm, zbits_v)
    pltpu.sync_copy(basis_hbm, basis_v)
    plsc.subcore_barrier()

    iota = lax.broadcasted_iota(jnp.int32, (16,), 0)
    iota2 = iota * 2

    def chunk(g, carry):
        base = wid * PER_W + g * C
        pltpu.sync_copy(syn_hbm.at[pl.ds(base * 2, C * 2)], stage_v)
        pltpu.sync_copy(w_hbm.at[pl.ds(base, C)], w_v)
        pltpu.sync_copy(sid_hbm.at[pl.ds(base, C)], sid_v)

        def vreg(v, carry2):
            l = v * 16
            ia = iota2 + l * 2
            post = plsc.load_gather(stage_v, [ia])
            pre = plsc.load_gather(stage_v, [ia + 1])
            zw = plsc.load_gather(zbits_v, [lax.shift_right_logical(pre, 5)])
            bit = lax.shift_right_logical(zw, pre & 31) & 1
            zf = bit.astype(jnp.float32)
            wv = w_v[pl.ds(l, 16)]
            wz = wv * zf
            sv = sid_v[pl.ds(l, 16)]
            sb = sv * N_BASIS
            post_v[pl.ds(l, 16)] = post
            rows = iota + l
            for r in range(N_BASIS):
                br = plsc.load_gather(basis_v, [sb + r])
                val = wz * br
                plsc.store_scatter(
                    vals_v, [rows, jnp.full((16,), r, jnp.int32)], val)
            return carry2

        lax.fori_loop(0, VPC, vreg, 0)
        pltpu.sync_copy(vals_v, acc.at[post_v], add=True)
        return carry

    lax.fori_loop(0, NCHUNK, chunk, 0)
    plsc.subcore_barrier()
    pltpu.sync_copy(acc.at[pl.ds(r0, STRIPE)],
                    out_hbm.at[c, pl.ds(r0, STRIPE)])


@functools.partial(
    pl.kernel,
    out_type=jax.ShapeDtypeStruct((NC, N_POST, N_BASIS), jnp.float32),
    mesh=plsc.VectorSubcoreMesh(core_axis_name="c", subcore_axis_name="s",
                                num_cores=NC, num_subcores=NS),
    compiler_params=pltpu.CompilerParams(needs_layout_passes=False,
                                         use_tc_tiling_on_sc=False),
    scratch_types=[
        pltpu.VMEM_SHARED((N_POST, N_BASIS), jnp.float32),
        pltpu.VMEM((NWORDS,), jnp.int32),
        pltpu.VMEM((512 * N_BASIS,), jnp.float32),
        pltpu.VMEM((2 * C,), jnp.int32),
        pltpu.VMEM((C,), jnp.float32),
        pltpu.VMEM((C,), jnp.int32),
        pltpu.VMEM((C,), jnp.int32),
        pltpu.VMEM((C, N_BASIS), jnp.float32),
    ],
)
def _sc_kernel(zbits_hbm, syn_hbm, w_hbm, basis_hbm, sid_hbm, zeros_hbm,
               out_hbm, acc, zbits_v, basis_v, stage_v, w_v, sid_v, post_v,
               vals_v):
    _sc_body(zbits_hbm, syn_hbm, w_hbm, basis_hbm, sid_hbm, zeros_hbm,
             out_hbm, acc, zbits_v, basis_v, stage_v, w_v, sid_v, post_v,
             vals_v)


def _tc_add(p_ref, o_ref):
    o_ref[:] = p_ref[0] + p_ref[1]


def kernel(rec_z_buf, synapse_indices, weight_values, synaptic_basis_weights,
           syn_ids):
    z_pad = jnp.pad(rec_z_buf.reshape(N_PRE), (0, NWORDS * 32 - N_PRE))
    zbits = pl.pallas_call(
        _pack_body,
        out_shape=jax.ShapeDtypeStruct((NWORDS,), jnp.int32),
    )(z_pad.reshape(NWORDS, 32))

    syn_flat = synapse_indices.reshape(2 * N_SYN)
    basis_flat = synaptic_basis_weights.reshape(512 * N_BASIS)
    zeros = jnp.zeros((N_POST, N_BASIS), jnp.float32)
    partials = _sc_kernel(zbits, syn_flat, weight_values, basis_flat,
                          syn_ids, zeros)
    p3 = partials.reshape(NC, 4000, 125)
    out = pl.pallas_call(
        _tc_add,
        out_shape=jax.ShapeDtypeStruct((4000, 125), jnp.float32),
    )(p3)
    return out.reshape(N_POST, N_BASIS)


# speedup vs baseline: 95.9230x; 8.1275x over previous
import functools

import jax
import jax.numpy as jnp
from jax import lax
from jax.experimental import pallas as pl
from jax.experimental.pallas import tpu as pltpu
from jax.experimental.pallas import tpu_sc as plsc

N_PRE = 100000
N_POST = 100000
N_SYN = 6400000
N_BASIS = 5

NC = 2
NS = 16
NW = NC * NS

PER_W = N_SYN // NW
C = 2000
NCHUNK = PER_W // C
VPC = C // 16

NWORDS = 3200

STRIPE = 6256
LAST_START = N_POST - STRIPE


def _pack_body(z_ref, o_ref):
    shifts = lax.broadcasted_iota(jnp.int32, (NWORDS, 32), 1)
    bits = jnp.where(z_ref[:] > 0.0, lax.shift_left(jnp.int32(1), shifts), 0)
    o_ref[:] = jnp.sum(bits, axis=1)


def _sc_body(zbits_hbm, post_hbm, pre_hbm, w_hbm, basis_hbm, sid_hbm,
             zeros_hbm, out_hbm, acc, zbits_v, basis_v, post_v, pre_v, w_v,
             sid_v, vals_v):
    c = lax.axis_index("c")
    s = lax.axis_index("s")
    wid = s * NC + c
    r0 = pl.multiple_of(jnp.where(s == NS - 1, LAST_START, s * STRIPE), 8)

    pltpu.sync_copy(zeros_hbm.at[pl.ds(r0, STRIPE)],
                    acc.at[pl.ds(r0, STRIPE)])
    pltpu.sync_copy(zbits_hbm, zbits_v)
    pltpu.sync_copy(basis_hbm, basis_v)
    plsc.subcore_barrier()

    iota = lax.broadcasted_iota(jnp.int32, (16,), 0)

    def chunk(g, carry):
        base = wid * PER_W + g * C
        pltpu.sync_copy(post_hbm.at[pl.ds(base, C)], post_v)
        pltpu.sync_copy(pre_hbm.at[pl.ds(base, C)], pre_v)
        pltpu.sync_copy(w_hbm.at[pl.ds(base, C)], w_v)
        pltpu.sync_copy(sid_hbm.at[pl.ds(base, C)], sid_v)

        def vreg(v, carry2):
            l = v * 16
            pre = pre_v[pl.ds(l, 16)]
            zw = plsc.load_gather(zbits_v, [lax.shift_right_logical(pre, 5)])
            bit = lax.shift_right_logical(zw, pre & 31) & 1
            zf = bit.astype(jnp.float32)
            wv = w_v[pl.ds(l, 16)]
            wz = wv * zf
            sv = sid_v[pl.ds(l, 16)]
            sb = sv * N_BASIS
            rows = iota + l
            for r in range(N_BASIS):
                br = plsc.load_gather(basis_v, [sb + r])
                val = wz * br
                plsc.store_scatter(
                    vals_v, [rows, jnp.full((16,), r, jnp.int32)], val)
            return carry2

        lax.fori_loop(0, VPC, vreg, 0)
        pltpu.sync_copy(vals_v, acc.at[post_v], add=True)
        return carry

    lax.fori_loop(0, NCHUNK, chunk, 0)
    plsc.subcore_barrier()
    pltpu.sync_copy(acc.at[pl.ds(r0, STRIPE)],
                    out_hbm.at[c, pl.ds(r0, STRIPE)])


@functools.partial(
    pl.kernel,
    out_type=jax.ShapeDtypeStruct((NC, N_POST, N_BASIS), jnp.float32),
    mesh=plsc.VectorSubcoreMesh(core_axis_name="c", subcore_axis_name="s",
                                num_cores=NC, num_subcores=NS),
    compiler_params=pltpu.CompilerParams(needs_layout_passes=False,
                                         use_tc_tiling_on_sc=False),
    scratch_types=[
        pltpu.VMEM_SHARED((N_POST, N_BASIS), jnp.float32),
        pltpu.VMEM((NWORDS,), jnp.int32),
        pltpu.VMEM((512 * N_BASIS,), jnp.float32),
        pltpu.VMEM((C,), jnp.int32),
        pltpu.VMEM((C,), jnp.int32),
        pltpu.VMEM((C,), jnp.float32),
        pltpu.VMEM((C,), jnp.int32),
        pltpu.VMEM((C, N_BASIS), jnp.float32),
    ],
)
def _sc_kernel(zbits_hbm, post_hbm, pre_hbm, w_hbm, basis_hbm, sid_hbm,
               zeros_hbm, out_hbm, acc, zbits_v, basis_v, post_v, pre_v, w_v,
               sid_v, vals_v):
    _sc_body(zbits_hbm, post_hbm, pre_hbm, w_hbm, basis_hbm, sid_hbm,
             zeros_hbm, out_hbm, acc, zbits_v, basis_v, post_v, pre_v, w_v,
             sid_v, vals_v)


def _tc_add(p_ref, o_ref):
    o_ref[:] = p_ref[0] + p_ref[1]


def kernel(rec_z_buf, synapse_indices, weight_values, synaptic_basis_weights,
           syn_ids):
    z_pad = jnp.pad(rec_z_buf.reshape(N_PRE), (0, NWORDS * 32 - N_PRE))
    zbits = pl.pallas_call(
        _pack_body,
        out_shape=jax.ShapeDtypeStruct((NWORDS,), jnp.int32),
    )(z_pad.reshape(NWORDS, 32))

    sp = synapse_indices.T
    post_arr = sp[0]
    pre_arr = sp[1]
    basis_flat = synaptic_basis_weights.reshape(512 * N_BASIS)
    zeros = jnp.zeros((N_POST, N_BASIS), jnp.float32)
    partials = _sc_kernel(zbits, post_arr, pre_arr, weight_values, basis_flat,
                          syn_ids, zeros)
    p3 = partials.reshape(NC, 4000, 125)
    out = pl.pallas_call(
        _tc_add,
        out_shape=jax.ShapeDtypeStruct((4000, 125), jnp.float32),
    )(p3)
    return out.reshape(N_POST, N_BASIS)
